# baseline (device time: 16247 ns/iter reference)
import jax
import jax.numpy as jnp
from jax import lax
from jax.experimental import pallas as pl
from jax.experimental.pallas import tpu as pltpu

N_DEV = 4
B = 2
SQ = 128
SKV = 128
H_LOC = 8
DH = 64
D = 512
DC = D // 2


def kernel(x, Wq, Wo, K_ext, V_ext):
    xf = x.reshape(B * SQ, D)
    kf = K_ext.reshape(B, SKV, H_LOC * DH)
    vf = V_ext.reshape(B, SKV, H_LOC * DH)

    def body(x_ref, wq_ref, wo_hbm, k_hbm, v_hbm, out_ref,
             attn_ref, send_ref, comm_ref, wo_ref, k_ref, v_ref,
             copy_sems, send_sems, recv_sems):
        my = lax.axis_index("i")
        partner1 = my ^ 1
        partner2 = 3 - my

        cp_wo = pltpu.make_async_copy(wo_hbm, wo_ref, copy_sems.at[0])
        cp_k = pltpu.make_async_copy(k_hbm, k_ref, copy_sems.at[1])
        cp_v = pltpu.make_async_copy(v_hbm, v_ref, copy_sems.at[2])
        cp_k.start()
        cp_v.start()
        cp_wo.start()

        barrier_sem = pltpu.get_barrier_semaphore()
        for nbr in (partner1, partner2):
            pl.semaphore_signal(
                barrier_sem, inc=1,
                device_id=(nbr,), device_id_type=pl.DeviceIdType.MESH,
            )
        pl.semaphore_wait(barrier_sem, 2)

        bf16 = jnp.bfloat16
        q_all = jnp.dot(x_ref[...].astype(bf16), wq_ref[...].astype(bf16),
                        preferred_element_type=jnp.float32)
        q3 = q_all.astype(bf16).reshape(B, SQ, D)
        cp_k.wait()
        cp_v.wait()
        kbf = k_ref[...].astype(bf16)
        vbf = v_ref[...].astype(bf16)
        for h in range(H_LOC):
            c = slice(h * DH, (h + 1) * DH)
            qh = q3[:, :, c]
            s = lax.dot_general(
                qh, kbf[:, :, c], (((2,), (2,)), ((0,), (0,))),
                preferred_element_type=jnp.float32) * 0.125
            p = jnp.exp(s)
            l = jnp.sum(p, axis=2, keepdims=True)
            o = lax.dot_general(
                p.astype(bf16), vbf[:, :, c], (((2,), (1,)), ((0,), (0,))),
                preferred_element_type=jnp.float32) / l
            attn_ref[:, c] = o.astype(bf16).reshape(B * SQ, DH)

        cp_wo.wait()
        out_ref[...] = jnp.dot(attn_ref[...], wo_ref[...].astype(bf16),
                               preferred_element_type=jnp.float32)

        for phase in range(2):
            for c in range(2):
                send_ref[c] = out_ref[:, c * DC:(c + 1) * DC].astype(bf16)
            rdmas = []
            for c in range(2):
                partner = partner1 if c == phase else partner2
                rdma = pltpu.make_async_remote_copy(
                    src_ref=send_ref.at[c],
                    dst_ref=comm_ref.at[phase, c],
                    send_sem=send_sems.at[phase, c],
                    recv_sem=recv_sems.at[phase, c],
                    device_id=(partner,),
                    device_id_type=pl.DeviceIdType.MESH,
                )
                rdma.start()
                rdmas.append(rdma)
            for c in range(2):
                rdmas[c].wait()
                out_ref[:, c * DC:(c + 1) * DC] += (
                    comm_ref[phase, c].astype(jnp.float32))

    out = pl.pallas_call(
        body,
        out_shape=jax.ShapeDtypeStruct((B * SQ, D), jnp.float32),
        in_specs=[
            pl.BlockSpec(memory_space=pltpu.VMEM),
            pl.BlockSpec(memory_space=pltpu.VMEM),
            pl.BlockSpec(memory_space=pl.ANY),
            pl.BlockSpec(memory_space=pl.ANY),
            pl.BlockSpec(memory_space=pl.ANY),
        ],
        out_specs=pl.BlockSpec(memory_space=pltpu.VMEM),
        scratch_shapes=[
            pltpu.VMEM((B * SQ, D), jnp.bfloat16),
            pltpu.VMEM((2, B * SQ, DC), jnp.bfloat16),
            pltpu.VMEM((2, 2, B * SQ, DC), jnp.bfloat16),
            pltpu.VMEM((D, D), jnp.float32),
            pltpu.VMEM((B, SKV, H_LOC * DH), jnp.float32),
            pltpu.VMEM((B, SKV, H_LOC * DH), jnp.float32),
            pltpu.SemaphoreType.DMA((3,)),
            pltpu.SemaphoreType.DMA((2, 2)),
            pltpu.SemaphoreType.DMA((2, 2)),
        ],
        compiler_params=pltpu.CompilerParams(collective_id=0),
    )(xf, Wq, Wo, kf, vf)
    return out.reshape(B, SQ, D)


# device time: 15592 ns/iter; 1.0420x vs baseline; 1.0420x over previous
import jax
import jax.numpy as jnp
from jax import lax
from jax.experimental import pallas as pl
from jax.experimental.pallas import tpu as pltpu

N_DEV = 4
B = 2
SQ = 128
SKV = 128
H_LOC = 8
DH = 64
D = 512
DC = D // 2


def kernel(x, Wq, Wo, K_ext, V_ext):
    xf = x.reshape(B * SQ, D)
    kf = K_ext.reshape(B, SKV, H_LOC * DH)
    vf = V_ext.reshape(B, SKV, H_LOC * DH)

    def body(x_ref, wq_ref, wo_ref, k_ref, v_ref, out_ref,
             attn_ref, send_ref, comm_ref, send_sems, recv_sems):
        my = lax.axis_index("i")
        partner1 = my ^ 1
        partner2 = 3 - my

        barrier_sem = pltpu.get_barrier_semaphore()
        for nbr in (partner1, partner2):
            pl.semaphore_signal(
                barrier_sem, inc=1,
                device_id=(nbr,), device_id_type=pl.DeviceIdType.MESH,
            )

        bf16 = jnp.bfloat16
        q_all = jnp.dot((x_ref[...] * 0.125).astype(bf16),
                        wq_ref[...].astype(bf16),
                        preferred_element_type=jnp.float32)
        q3 = q_all.astype(bf16).reshape(B, SQ, D)
        kbf = k_ref[...].astype(bf16)
        vbf = v_ref[...].astype(bf16)
        for h in range(H_LOC):
            c = slice(h * DH, (h + 1) * DH)
            qh = q3[:, :, c]
            s = lax.dot_general(
                qh, kbf[:, :, c], (((2,), (2,)), ((0,), (0,))),
                preferred_element_type=jnp.float32)
            p = jnp.exp(s)
            l = jnp.sum(p, axis=2, keepdims=True)
            o = lax.dot_general(
                p.astype(bf16), vbf[:, :, c], (((2,), (1,)), ((0,), (0,))),
                preferred_element_type=jnp.float32) / l
            attn_ref[:, c] = o.astype(bf16).reshape(B * SQ, DH)

        out_ref[...] = jnp.dot(attn_ref[...], wo_ref[...].astype(bf16),
                               preferred_element_type=jnp.float32)

        def exchange(phase, c):
            partner = partner1 if c == phase else partner2
            send_ref[phase, c] = out_ref[:, c * DC:(c + 1) * DC].astype(bf16)
            rdma = pltpu.make_async_remote_copy(
                src_ref=send_ref.at[phase, c],
                dst_ref=comm_ref.at[phase, c],
                send_sem=send_sems.at[phase, c],
                recv_sem=recv_sems.at[phase, c],
                device_id=(partner,),
                device_id_type=pl.DeviceIdType.MESH,
            )
            rdma.start()
            return rdma

        def accumulate(rdma, phase, c):
            rdma.wait()
            out_ref[:, c * DC:(c + 1) * DC] += (
                comm_ref[phase, c].astype(jnp.float32))

        pl.semaphore_wait(barrier_sem, 2)
        p0 = [exchange(0, c) for c in range(2)]
        p1 = [None, None]
        for c in range(2):
            accumulate(p0[c], 0, c)
            p1[c] = exchange(1, c)
        for c in range(2):
            accumulate(p1[c], 1, c)

    out = pl.pallas_call(
        body,
        out_shape=jax.ShapeDtypeStruct((B * SQ, D), jnp.float32),
        in_specs=[pl.BlockSpec(memory_space=pltpu.VMEM)] * 5,
        out_specs=pl.BlockSpec(memory_space=pltpu.VMEM),
        scratch_shapes=[
            pltpu.VMEM((B * SQ, D), jnp.bfloat16),
            pltpu.VMEM((2, 2, B * SQ, DC), jnp.bfloat16),
            pltpu.VMEM((2, 2, B * SQ, DC), jnp.bfloat16),
            pltpu.SemaphoreType.DMA((2, 2)),
            pltpu.SemaphoreType.DMA((2, 2)),
        ],
        compiler_params=pltpu.CompilerParams(collective_id=0),
    )(xf, Wq, Wo, kf, vf)
    return out.reshape(B, SQ, D)


# device time: 15479 ns/iter; 1.0496x vs baseline; 1.0073x over previous
import jax
import jax.numpy as jnp
from jax import lax
from jax.experimental import pallas as pl
from jax.experimental.pallas import tpu as pltpu

N_DEV = 4
B = 2
SQ = 128
SKV = 128
H_LOC = 8
DH = 64
D = 512
DC = D // 2


def kernel(x, Wq, Wo, K_ext, V_ext):
    xf = x.reshape(B * SQ, D)
    kf = K_ext.reshape(B, SKV, H_LOC * DH)
    vf = V_ext.reshape(B, SKV, H_LOC * DH)

    def body(x_ref, wq_ref, wo_ref, k_ref, v_ref, out_ref,
             attn_ref, acc_ref, comm_ref, send_sems, recv_sems):
        my = lax.axis_index("i")
        partner1 = my ^ 1
        partner2 = 3 - my

        barrier_sem = pltpu.get_barrier_semaphore()
        for nbr in (partner1, partner2):
            pl.semaphore_signal(
                barrier_sem, inc=1,
                device_id=(nbr,), device_id_type=pl.DeviceIdType.MESH,
            )

        bf16 = jnp.bfloat16
        q_all = jnp.dot((x_ref[...] * 0.125).astype(bf16),
                        wq_ref[...].astype(bf16),
                        preferred_element_type=jnp.float32)
        q3 = q_all.astype(bf16).reshape(B, SQ, D)
        kbf = k_ref[...].astype(bf16)
        vbf = v_ref[...].astype(bf16)
        for h in range(H_LOC):
            c = slice(h * DH, (h + 1) * DH)
            qh = q3[:, :, c]
            s = lax.dot_general(
                qh, kbf[:, :, c], (((2,), (2,)), ((0,), (0,))),
                preferred_element_type=jnp.float32)
            p = jnp.exp(s)
            l = jnp.sum(p, axis=2, keepdims=True)
            o = lax.dot_general(
                p.astype(bf16), vbf[:, :, c], (((2,), (1,)), ((0,), (0,))),
                preferred_element_type=jnp.float32) / l
            attn_ref[:, c] = o.astype(bf16).reshape(B * SQ, DH)

        wo_bf = wo_ref[...].astype(bf16)
        for c in range(2):
            acc_ref[c] = jnp.dot(
                attn_ref[...], wo_bf[:, c * DC:(c + 1) * DC],
                preferred_element_type=jnp.float32).astype(bf16)

        def exchange(phase, c):
            partner = partner1 if c == phase else partner2
            rdma = pltpu.make_async_remote_copy(
                src_ref=acc_ref.at[c],
                dst_ref=comm_ref.at[phase, c],
                send_sem=send_sems.at[phase, c],
                recv_sem=recv_sems.at[phase, c],
                device_id=(partner,),
                device_id_type=pl.DeviceIdType.MESH,
            )
            rdma.start()
            return rdma

        def accumulate(rdma, phase, c):
            rdma.wait()
            acc_ref[c] += comm_ref[phase, c]

        pl.semaphore_wait(barrier_sem, 2)
        p0 = [exchange(0, c) for c in range(2)]
        p1 = [None, None]
        for c in range(2):
            accumulate(p0[c], 0, c)
            p1[c] = exchange(1, c)
        for c in range(2):
            accumulate(p1[c], 1, c)
            out_ref[:, c * DC:(c + 1) * DC] = acc_ref[c].astype(jnp.float32)

    out = pl.pallas_call(
        body,
        out_shape=jax.ShapeDtypeStruct((B * SQ, D), jnp.float32),
        in_specs=[pl.BlockSpec(memory_space=pltpu.VMEM)] * 5,
        out_specs=pl.BlockSpec(memory_space=pltpu.VMEM),
        scratch_shapes=[
            pltpu.VMEM((B * SQ, D), jnp.bfloat16),
            pltpu.VMEM((2, B * SQ, DC), jnp.bfloat16),
            pltpu.VMEM((2, 2, B * SQ, DC), jnp.bfloat16),
            pltpu.SemaphoreType.DMA((2, 2)),
            pltpu.SemaphoreType.DMA((2, 2)),
        ],
        compiler_params=pltpu.CompilerParams(collective_id=0),
    )(xf, Wq, Wo, kf, vf)
    return out.reshape(B, SQ, D)
